# Initial kernel scaffold; baseline (speedup 1.0000x reference)
import jax
import jax.numpy as jnp
from jax.experimental import pallas as pl


def kernel(node_features, edge_index, edge_attr, membership, candidate_batch, mask, W_in, b_in, W_edge, b_edge, W1_0, b1_0, W2_0, b2_0, W1_1, b1_1, W2_1, b2_1, W1_2, b1_2, W2_2, b2_2, Wp1, bp1, Wp2, bp2, Wp3, bp3):
    raise NotImplementedError("write your pallas kernel here")


# speedup vs baseline: 1.8668x; 1.8668x over previous
import functools

import jax
import jax.numpy as jnp
from jax import lax
from jax.experimental import pallas as pl
from jax.experimental.pallas import tpu as pltpu
from jax.experimental.pallas import tpu_sc as plsc

N = 10000
E = 320000
H = 128
NF = 512
NG = 8

NEG_BIG = -1000000000.0


_NTILES = 32
_EPT = E // _NTILES
_C = 80
_NCH = _EPT // _C
_NP = 10240
_RPS = _NP // 16
_ZR = 128

_sc_mesh = plsc.VectorSubcoreMesh(core_axis_name="c", subcore_axis_name="s")


@functools.partial(
    pl.kernel,
    out_type=jax.ShapeDtypeStruct((2, _NP, H), jnp.float32),
    mesh=_sc_mesh,
    scratch_types=[
        pltpu.VMEM_SHARED((_NP, H), jnp.float32),
        pltpu.VMEM((_C,), jnp.int32),
        pltpu.VMEM((_C,), jnp.int32),
        pltpu.VMEM((_C,), jnp.float32),
        pltpu.VMEM((_C, H), jnp.float32),
        pltpu.VMEM((H,), jnp.float32),
        pltpu.VMEM((H,), jnp.float32),
        pltpu.VMEM((_ZR, H), jnp.float32),
        pltpu.SemaphoreType.DMA,
    ],
)
def _sc_message(x_hbm, src_hbm, dst_hbm, ea_hbm, we_hbm, be_hbm, out_hbm,
                agg_sh, srcv, dstv, eav, rows, wv, bv, zbuf, sem):
    c = lax.axis_index("c")
    s = lax.axis_index("s")
    tile = c * 16 + s

    zero16 = jnp.zeros((16,), jnp.float32)

    def _zrow(i, carry):
        for j in range(H // 16):
            zbuf[i, pl.ds(j * 16, 16)] = zero16
        return carry

    lax.fori_loop(0, _ZR, _zrow, 0, unroll=False)

    def _zcopy(k, carry):
        pltpu.sync_copy(zbuf, agg_sh.at[pl.ds(s * _RPS + k * _ZR, _ZR)])
        return carry

    lax.fori_loop(0, _RPS // _ZR, _zcopy, 0, unroll=False)

    pltpu.sync_copy(we_hbm, wv)
    pltpu.sync_copy(be_hbm, bv)

    plsc.subcore_barrier()

    def _chunk(t, carry):
        base = tile * _EPT + t * _C
        pltpu.sync_copy(src_hbm.at[pl.ds(base, _C)], srcv)
        pltpu.sync_copy(dst_hbm.at[pl.ds(base, _C)], dstv)
        pltpu.sync_copy(ea_hbm.at[pl.ds(base, _C)], eav)
        pltpu.async_copy(x_hbm.at[srcv], rows, sem).wait()

        def _grp(g, ecarry):
            ea16 = eav[pl.ds(g * 16, 16)]
            for k in range(16):
                a = ea16[k]
                i = g * 16 + k
                for j in range(H // 16):
                    sl = pl.ds(j * 16, 16)
                    rows[i, sl] = jnp.maximum(
                        rows[i, sl] + (a * wv[sl] + bv[sl]), 0.0)
            return ecarry

        lax.fori_loop(0, _C // 16, _grp, 0, unroll=False)

        pltpu.sync_copy(rows, agg_sh.at[dstv], add=True)
        return carry

    lax.fori_loop(0, _NCH, _chunk, 0, unroll=False)

    plsc.subcore_barrier()

    def _out(k, carry):
        sl = pl.ds(s * _RPS + k * _ZR, _ZR)
        pltpu.sync_copy(agg_sh.at[sl], out_hbm.at[c, sl])
        return carry

    lax.fori_loop(0, _RPS // _ZR, _out, 0, unroll=False)


_RB = 2000
_NRB = N // _RB


def _proj_body(x_ref, w_ref, b_ref, o_ref):
    o_ref[...] = (
        jnp.dot(x_ref[...], w_ref[...], preferred_element_type=jnp.float32)
        + b_ref[...]
    )


def _tc_project(nf, W_in, b_in):
    return pl.pallas_call(
        _proj_body,
        grid=(_NRB,),
        in_specs=[
            pl.BlockSpec((_RB, H), lambda i: (i, 0)),
            pl.BlockSpec((H, H), lambda i: (0, 0)),
            pl.BlockSpec((1, H), lambda i: (0, 0)),
        ],
        out_specs=pl.BlockSpec((_RB, H), lambda i: (i, 0)),
        out_shape=jax.ShapeDtypeStruct((N, H), jnp.float32),
    )(nf, W_in, b_in.reshape(1, H))


def _layer_body(x_ref, a_ref, w1_ref, b1_ref, w2_ref, b2_ref, o_ref):
    h = x_ref[...] + a_ref[0] + a_ref[1]
    h = jnp.maximum(
        jnp.dot(h, w1_ref[...], preferred_element_type=jnp.float32)
        + b1_ref[...], 0.0)
    h = jnp.dot(h, w2_ref[...], preferred_element_type=jnp.float32) + b2_ref[...]
    o_ref[...] = jnp.maximum(h, 0.0)


def _tc_layer(x, agg2, W1, b1, W2, b2):
    return pl.pallas_call(
        _layer_body,
        grid=(_NRB,),
        in_specs=[
            pl.BlockSpec((_RB, H), lambda i: (i, 0)),
            pl.BlockSpec((2, _RB, H), lambda i: (0, i, 0)),
            pl.BlockSpec((H, H), lambda i: (0, 0)),
            pl.BlockSpec((1, H), lambda i: (0, 0)),
            pl.BlockSpec((H, H), lambda i: (0, 0)),
            pl.BlockSpec((1, H), lambda i: (0, 0)),
        ],
        out_specs=pl.BlockSpec((_RB, H), lambda i: (i, 0)),
        out_shape=jax.ShapeDtypeStruct((N, H), jnp.float32),
    )(x, agg2, W1, b1.reshape(1, H), W2, b2.reshape(1, H))


def _head_body(x_ref, mem_ref, cb_ref, mask_ref,
               wp1_ref, bp1_ref, wp2_ref, bp2_ref, wp3_ref, bp3_ref,
               o_ref, sums, cnts):
    i = pl.program_id(0)

    @pl.when(i == 0)
    def _init():
        sums[...] = jnp.zeros_like(sums)
        cnts[...] = jnp.zeros_like(cnts)

    mem = mem_ref[0, 0, :]
    seg = lax.broadcasted_iota(jnp.int32, (NF, _RB), 0)
    onehot = (seg == mem[None, :]).astype(jnp.float32)
    sums[...] += jnp.dot(onehot, x_ref[...],
                         preferred_element_type=jnp.float32)
    cnts[...] += jnp.sum(onehot, axis=1, keepdims=True)

    @pl.when(i == _NRB - 1)
    def _final():
        z = sums[...] / jnp.maximum(cnts[...], 1.0)
        cb = cb_ref[0, 0, :]
        gi = lax.broadcasted_iota(jnp.int32, (NG, NF), 0)
        og = (gi == cb[None, :]).astype(jnp.float32)
        csum = jnp.dot(og, z, preferred_element_type=jnp.float32)
        ccnt = jnp.sum(og, axis=1, keepdims=True)
        ctx_per = csum / jnp.maximum(ccnt, 1.0)
        gi2 = lax.broadcasted_iota(jnp.int32, (NF, NG), 1)
        ogt = (gi2 == cb[:, None]).astype(jnp.float32)
        ctx = jnp.dot(ogt, ctx_per, preferred_element_type=jnp.float32)
        p = jnp.maximum(
            jnp.dot(z, wp1_ref[0], preferred_element_type=jnp.float32)
            + jnp.dot(ctx, wp1_ref[1], preferred_element_type=jnp.float32)
            + bp1_ref[...], 0.0)
        p = jnp.maximum(
            jnp.dot(p, wp2_ref[...], preferred_element_type=jnp.float32)
            + bp2_ref[...], 0.0)
        logits = jnp.sum(p * wp3_ref[...].reshape(1, H), axis=1,
                         keepdims=True) + bp3_ref[0, 0]
        m = mask_ref[...]
        logits = jnp.where(m > 0.5, logits, NEG_BIG)
        o_ref[...] = jnp.broadcast_to(logits, (NF, H))


def _tc_head(x, membership, candidate_batch, mask,
             Wp1, bp1, Wp2, bp2, Wp3, bp3):
    mem3 = membership.astype(jnp.int32).reshape(_NRB, 1, _RB)
    cb3 = candidate_batch.astype(jnp.int32).reshape(1, 1, NF)
    maskf = mask.astype(jnp.float32).reshape(NF, 1)
    wp1_3 = Wp1.reshape(2, H, H)
    out = pl.pallas_call(
        _head_body,
        grid=(_NRB,),
        in_specs=[
            pl.BlockSpec((_RB, H), lambda i: (i, 0)),
            pl.BlockSpec((1, 1, _RB), lambda i: (i, 0, 0)),
            pl.BlockSpec((1, 1, NF), lambda i: (0, 0, 0)),
            pl.BlockSpec((NF, 1), lambda i: (0, 0)),
            pl.BlockSpec((2, H, H), lambda i: (0, 0, 0)),
            pl.BlockSpec((1, H), lambda i: (0, 0)),
            pl.BlockSpec((H, H), lambda i: (0, 0)),
            pl.BlockSpec((1, H), lambda i: (0, 0)),
            pl.BlockSpec((1, H), lambda i: (0, 0)),
            pl.BlockSpec((1, 1), lambda i: (0, 0)),
        ],
        out_specs=pl.BlockSpec((NF, H), lambda i: (0, 0)),
        out_shape=jax.ShapeDtypeStruct((NF, H), jnp.float32),
        scratch_shapes=[
            pltpu.VMEM((NF, H), jnp.float32),
            pltpu.VMEM((NF, 1), jnp.float32),
        ],
    )(x, mem3, cb3, maskf, wp1_3, bp1.reshape(1, H), Wp2, bp2.reshape(1, H),
      Wp3.reshape(1, H), bp3.reshape(1, 1))
    return out[:, 0]


def kernel(node_features, edge_index, edge_attr, membership, candidate_batch,
           mask, W_in, b_in, W_edge, b_edge,
           W1_0, b1_0, W2_0, b2_0,
           W1_1, b1_1, W2_1, b2_1,
           W1_2, b1_2, W2_2, b2_2,
           Wp1, bp1, Wp2, bp2, Wp3, bp3):
    src = edge_index[0].astype(jnp.int32)
    dst = edge_index[1].astype(jnp.int32)
    ea = edge_attr[:, 0].astype(jnp.float32)
    we = W_edge[0].astype(jnp.float32)
    be = b_edge.astype(jnp.float32)

    x = _tc_project(node_features.astype(jnp.float32), W_in, b_in)
    for (W1, b1, W2, b2) in ((W1_0, b1_0, W2_0, b2_0),
                             (W1_1, b1_1, W2_1, b2_1),
                             (W1_2, b1_2, W2_2, b2_2)):
        agg2 = _sc_message(x, src, dst, ea, we, be)
        x = _tc_layer(x, agg2, W1, b1, W2, b2)
    return _tc_head(x, membership, candidate_batch, mask,
                    Wp1, bp1, Wp2, bp2, Wp3, bp3)


# speedup vs baseline: 7.7157x; 4.1331x over previous
import functools

import jax
import jax.numpy as jnp
from jax import lax
from jax.experimental import pallas as pl
from jax.experimental.pallas import tpu as pltpu
from jax.experimental.pallas import tpu_sc as plsc

N = 10000
E = 320000
H = 128
NF = 512
NG = 8

NEG_BIG = -1000000000.0


_NTILES = 32
_EPT = E // _NTILES
_C = 80
_NCH = _EPT // _C
_NB = 3
_NO = 41
_NP = 10240
_RPS = _NP // 16

_sc_mesh = plsc.VectorSubcoreMesh(core_axis_name="c", subcore_axis_name="s")


@functools.partial(
    pl.kernel,
    out_type=jax.ShapeDtypeStruct((2, _NP, H), jnp.float32),
    mesh=_sc_mesh,
    scratch_types=[
        pltpu.VMEM_SHARED((_NP, H), jnp.float32),
        [pltpu.VMEM((_C,), jnp.int32) for _ in range(_NB)],
        [pltpu.VMEM((_C,), jnp.int32) for _ in range(_NB)],
        [pltpu.VMEM((_C,), jnp.float32) for _ in range(_NB)],
        [pltpu.VMEM((_C, H), jnp.float32) for _ in range(_NB)],
        pltpu.VMEM((H,), jnp.float32),
        pltpu.VMEM((H,), jnp.float32),
        [pltpu.SemaphoreType.DMA for _ in range(_NB)],
        [pltpu.SemaphoreType.DMA for _ in range(_NB)],
    ],
)
def _sc_message(x_hbm, src_hbm, dst_hbm, ea_hbm, we_hbm, be_hbm, out_hbm,
                agg_sh, srcv, dstv, eav, rows, wv, bv, gsem, isem):
    c = lax.axis_index("c")
    s = lax.axis_index("s")
    tile = c * 16 + s
    ebase = tile * _EPT

    pltpu.sync_copy(we_hbm, wv)
    pltpu.sync_copy(be_hbm, bv)

    zero16 = jnp.zeros((16,), jnp.float32)

    def _zrow(i, carry):
        for j in range(H // 16):
            rows[0][i, pl.ds(j * 16, 16)] = zero16
        return carry

    lax.fori_loop(0, _C, _zrow, 0, unroll=False)

    def _zcopy(k, carry):
        pltpu.sync_copy(rows[0], agg_sh.at[pl.ds(s * _RPS + k * _C, _C)])
        return carry

    lax.fori_loop(0, _RPS // _C, _zcopy, 0, unroll=False)

    wregs = [wv[pl.ds(j * 16, 16)] for j in range(H // 16)]
    bregs = [bv[pl.ds(j * 16, 16)] for j in range(H // 16)]

    plsc.subcore_barrier()

    def _idx_copies(t, b):
        base = ebase + t * _C
        return (
            pltpu.make_async_copy(src_hbm.at[pl.ds(base, _C)], srcv[b], isem[b]),
            pltpu.make_async_copy(dst_hbm.at[pl.ds(base, _C)], dstv[b], isem[b]),
            pltpu.make_async_copy(ea_hbm.at[pl.ds(base, _C)], eav[b], isem[b]),
        )

    def _gather(b):
        return pltpu.make_async_copy(x_hbm.at[srcv[b]], rows[b], gsem[b])

    for b in range(_NB):
        for cp in _idx_copies(b, b):
            cp.start()
    for cp in _idx_copies(0, 0):
        cp.wait()
    _gather(0).start()

    def _slot(t, b, has_next, has_next3):
        _gather(b).wait()
        b1 = (b + 1) % _NB

        def _issue_next():
            for cp in _idx_copies(t + 1, b1):
                cp.wait()
            _gather(b1).start()

        if has_next is True:
            _issue_next()
        elif has_next is not False:
            pl.when(has_next)(_issue_next)

        def _grp(g, ecarry):
            ea16 = eav[b][pl.ds(g * 16, 16)]
            for k in range(16):
                a = ea16[k]
                i = g * 16 + k
                for j in range(H // 16):
                    sl = pl.ds(j * 16, 16)
                    rows[b][i, sl] = jnp.maximum(
                        rows[b][i, sl] + (a * wregs[j] + bregs[j]), 0.0)
            return ecarry

        lax.fori_loop(0, _C // 16, _grp, 0, unroll=False)

        pltpu.sync_copy(rows[b], agg_sh.at[dstv[b]], add=True)

        def _issue_idx3():
            for cp in _idx_copies(t + _NB, b):
                cp.start()

        if has_next3 is True:
            _issue_idx3()
        elif has_next3 is not False:
            pl.when(has_next3)(_issue_idx3)

    def _outer(o, carry):
        for b in range(_NB):
            t = o * _NB + b
            _slot(t, b, True, t + _NB < _NCH)
        return carry

    lax.fori_loop(0, _NO, _outer, 0, unroll=False)

    _slot(_NCH - 2, (_NCH - 2) % _NB, True, False)
    _slot(_NCH - 1, (_NCH - 1) % _NB, False, False)

    plsc.subcore_barrier()

    def _out(k, carry):
        sl = pl.ds(s * _RPS + k * _C, _C)
        pltpu.sync_copy(agg_sh.at[sl], out_hbm.at[c, sl])
        return carry

    lax.fori_loop(0, _RPS // _C, _out, 0, unroll=False)


_RB = 2000
_NRB = N // _RB


def _dot_bf16(a, b):
    return jnp.dot(a.astype(jnp.bfloat16), b.astype(jnp.bfloat16),
                   preferred_element_type=jnp.float32)


def _proj_body(x_ref, w_ref, b_ref, o_ref):
    o_ref[...] = _dot_bf16(x_ref[...], w_ref[...]) + b_ref[...]


def _tc_project(nf, W_in, b_in):
    return pl.pallas_call(
        _proj_body,
        grid=(_NRB,),
        in_specs=[
            pl.BlockSpec((_RB, H), lambda i: (i, 0)),
            pl.BlockSpec((H, H), lambda i: (0, 0)),
            pl.BlockSpec((1, H), lambda i: (0, 0)),
        ],
        out_specs=pl.BlockSpec((_RB, H), lambda i: (i, 0)),
        out_shape=jax.ShapeDtypeStruct((N, H), jnp.float32),
    )(nf, W_in, b_in.reshape(1, H))


def _layer_body(x_ref, a_ref, w1_ref, b1_ref, w2_ref, b2_ref, o_ref):
    h = x_ref[...] + a_ref[0] + a_ref[1]
    h = jnp.maximum(_dot_bf16(h, w1_ref[...]) + b1_ref[...], 0.0)
    h = _dot_bf16(h, w2_ref[...]) + b2_ref[...]
    o_ref[...] = jnp.maximum(h, 0.0)


def _tc_layer(x, agg2, W1, b1, W2, b2):
    return pl.pallas_call(
        _layer_body,
        grid=(_NRB,),
        in_specs=[
            pl.BlockSpec((_RB, H), lambda i: (i, 0)),
            pl.BlockSpec((2, _RB, H), lambda i: (0, i, 0)),
            pl.BlockSpec((H, H), lambda i: (0, 0)),
            pl.BlockSpec((1, H), lambda i: (0, 0)),
            pl.BlockSpec((H, H), lambda i: (0, 0)),
            pl.BlockSpec((1, H), lambda i: (0, 0)),
        ],
        out_specs=pl.BlockSpec((_RB, H), lambda i: (i, 0)),
        out_shape=jax.ShapeDtypeStruct((N, H), jnp.float32),
    )(x, agg2, W1, b1.reshape(1, H), W2, b2.reshape(1, H))


def _head_body(x_ref, mem_ref, cb_ref, mask_ref,
               wp1_ref, bp1_ref, wp2_ref, bp2_ref, wp3_ref, bp3_ref,
               o_ref, sums, cnts):
    i = pl.program_id(0)

    @pl.when(i == 0)
    def _init():
        sums[...] = jnp.zeros_like(sums)
        cnts[...] = jnp.zeros_like(cnts)

    mem = mem_ref[0, 0, :]
    seg = lax.broadcasted_iota(jnp.int32, (NF, _RB), 0)
    onehot = (seg == mem[None, :]).astype(jnp.float32)
    sums[...] += jnp.dot(onehot, x_ref[...],
                         preferred_element_type=jnp.float32, precision=lax.Precision.HIGHEST)
    cnts[...] += jnp.sum(onehot, axis=1, keepdims=True)

    @pl.when(i == _NRB - 1)
    def _final():
        z = sums[...] / jnp.maximum(cnts[...], 1.0)
        cb = cb_ref[0, 0, :]
        gi = lax.broadcasted_iota(jnp.int32, (NG, NF), 0)
        og = (gi == cb[None, :]).astype(jnp.float32)
        csum = jnp.dot(og, z, preferred_element_type=jnp.float32, precision=lax.Precision.HIGHEST)
        ccnt = jnp.sum(og, axis=1, keepdims=True)
        ctx_per = csum / jnp.maximum(ccnt, 1.0)
        gi2 = lax.broadcasted_iota(jnp.int32, (NF, NG), 1)
        ogt = (gi2 == cb[:, None]).astype(jnp.float32)
        ctx = jnp.dot(ogt, ctx_per, preferred_element_type=jnp.float32, precision=lax.Precision.HIGHEST)
        p = jnp.maximum(
            _dot_bf16(z, wp1_ref[0]) + _dot_bf16(ctx, wp1_ref[1])
            + bp1_ref[...], 0.0)
        p = jnp.maximum(_dot_bf16(p, wp2_ref[...]) + bp2_ref[...], 0.0)
        pb = p.astype(jnp.bfloat16).astype(jnp.float32)
        w3b = wp3_ref[...].reshape(1, H).astype(jnp.bfloat16).astype(jnp.float32)
        logits = jnp.sum(pb * w3b, axis=1, keepdims=True) + bp3_ref[0, 0]
        m = mask_ref[...]
        logits = jnp.where(m > 0.5, logits, NEG_BIG)
        o_ref[...] = jnp.broadcast_to(logits, (NF, H))


def _tc_head(x, membership, candidate_batch, mask,
             Wp1, bp1, Wp2, bp2, Wp3, bp3):
    mem3 = membership.astype(jnp.int32).reshape(_NRB, 1, _RB)
    cb3 = candidate_batch.astype(jnp.int32).reshape(1, 1, NF)
    maskf = mask.astype(jnp.float32).reshape(NF, 1)
    wp1_3 = Wp1.reshape(2, H, H)
    out = pl.pallas_call(
        _head_body,
        grid=(_NRB,),
        in_specs=[
            pl.BlockSpec((_RB, H), lambda i: (i, 0)),
            pl.BlockSpec((1, 1, _RB), lambda i: (i, 0, 0)),
            pl.BlockSpec((1, 1, NF), lambda i: (0, 0, 0)),
            pl.BlockSpec((NF, 1), lambda i: (0, 0)),
            pl.BlockSpec((2, H, H), lambda i: (0, 0, 0)),
            pl.BlockSpec((1, H), lambda i: (0, 0)),
            pl.BlockSpec((H, H), lambda i: (0, 0)),
            pl.BlockSpec((1, H), lambda i: (0, 0)),
            pl.BlockSpec((1, H), lambda i: (0, 0)),
            pl.BlockSpec((1, 1), lambda i: (0, 0)),
        ],
        out_specs=pl.BlockSpec((NF, H), lambda i: (0, 0)),
        out_shape=jax.ShapeDtypeStruct((NF, H), jnp.float32),
        scratch_shapes=[
            pltpu.VMEM((NF, H), jnp.float32),
            pltpu.VMEM((NF, 1), jnp.float32),
        ],
    )(x, mem3, cb3, maskf, wp1_3, bp1.reshape(1, H), Wp2, bp2.reshape(1, H),
      Wp3.reshape(1, H), bp3.reshape(1, 1))
    return out[:, 0]


def kernel(node_features, edge_index, edge_attr, membership, candidate_batch,
           mask, W_in, b_in, W_edge, b_edge,
           W1_0, b1_0, W2_0, b2_0,
           W1_1, b1_1, W2_1, b2_1,
           W1_2, b1_2, W2_2, b2_2,
           Wp1, bp1, Wp2, bp2, Wp3, bp3):
    src = edge_index[0].astype(jnp.int32)
    dst = edge_index[1].astype(jnp.int32)
    ea = edge_attr[:, 0].astype(jnp.float32)
    we = W_edge[0].astype(jnp.float32)
    be = b_edge.astype(jnp.float32)

    x = _tc_project(node_features.astype(jnp.float32), W_in, b_in)
    for (W1, b1, W2, b2) in ((W1_0, b1_0, W2_0, b2_0),
                             (W1_1, b1_1, W2_1, b2_1),
                             (W1_2, b1_2, W2_2, b2_2)):
        agg2 = _sc_message(x, src, dst, ea, we, be)
        x = _tc_layer(x, agg2, W1, b1, W2, b2)
    return _tc_head(x, membership, candidate_batch, mask,
                    Wp1, bp1, Wp2, bp2, Wp3, bp3)
